# baseline (speedup 1.0000x reference)
import dataclasses
import functools

import jax
import jax.numpy as jnp
from jax import lax
from jax.experimental import pallas as pl
from jax.experimental.pallas import tpu as pltpu
from jax.experimental.pallas import tpu_sc as plsc

K = 60
IN_MIN = 0.0
IN_MAX = 1.0
SCALE = (K - 1) / max(IN_MAX - IN_MIN, 1e-12)

LANES = 16
BLOCK_R = 40
BLOCK_C = 512
VEC_PER_ROW = BLOCK_C // LANES

SC_ROWS = 120
TC_BLOCK_R = 40
TC_BLOCK_C = 2048


def _spline_body(ctab):
    def body(x_vmem, o_vmem):
        @plsc.parallel_loop(0, BLOCK_R * VEC_PER_ROW, 1, unroll=8)
        def _(v):
            r = v >> 5
            c = (v & (VEC_PER_ROW - 1)) * LANES
            xv = x_vmem[r, pl.ds(c, LANES)]
            t = xv * SCALE
            i0 = t.astype(jnp.int32)
            alpha = t - i0.astype(jnp.float32)
            c0 = plsc.load_gather(ctab, [i0])
            c1 = plsc.load_gather(ctab, [i0 + 1])
            o_vmem[r, pl.ds(c, LANES)] = c0 + alpha * (c1 - c0)
    return body


def _tc_body(x_ref, c_ref, o_ref):
    tab = jnp.broadcast_to(c_ref[0, :][None, :], (TC_BLOCK_R, 128))
    t = x_ref[...] * SCALE
    i0 = t.astype(jnp.int32)
    alpha = t - i0.astype(jnp.float32)
    c0 = jnp.take_along_axis(tab, i0, axis=1)
    c1 = jnp.take_along_axis(tab, i0 + 1, axis=1)
    o_ref[...] = c0 + alpha * (c1 - c0)


@jax.jit
def kernel(x, coeffs):
    xt = x.T
    nr, nc = xt.shape
    coeffs_padded = jnp.pad(coeffs, (0, 64 - K))

    mesh = plsc.VectorSubcoreMesh(core_axis_name="c", subcore_axis_name="s")
    cp = pltpu.CompilerParams(use_tc_tiling_on_sc=True)
    if "needs_layout_passes" in pltpu.CompilerParams.__dataclass_fields__:
        cp = dataclasses.replace(cp, needs_layout_passes=False)

    @functools.partial(
        pl.kernel,
        out_type=jax.ShapeDtypeStruct((nr, nc), jnp.float32),
        mesh=mesh,
        scratch_types=[pltpu.VMEM((64,), jnp.float32)],
        compiler_params=cp,
    )
    def run(x_hbm, c_hbm, o_hbm, ctab):
        pltpu.sync_copy(c_hbm, ctab)
        pltpu.emit_pipeline(
            _spline_body(ctab),
            grid=(SC_ROWS // BLOCK_R, nc // BLOCK_C),
            in_specs=[pl.BlockSpec((BLOCK_R, BLOCK_C), lambda i, j: (i, j))],
            out_specs=[pl.BlockSpec((BLOCK_R, BLOCK_C), lambda i, j: (i, j))],
            core_axis_name=("c", "s"),
            dimension_semantics=(pltpu.PARALLEL, pltpu.PARALLEL),
        )(x_hbm, o_hbm)

    sc_out = run(xt, coeffs_padded)

    ctab_tc = jnp.pad(coeffs, (0, 128 - K)).reshape(1, 128)
    n_rb = SC_ROWS // TC_BLOCK_R
    tc_out = pl.pallas_call(
        _tc_body,
        grid=((nr - SC_ROWS) // TC_BLOCK_R, nc // TC_BLOCK_C),
        in_specs=[
            pl.BlockSpec((TC_BLOCK_R, TC_BLOCK_C),
                         lambda i, j: (n_rb + i, j)),
            pl.BlockSpec((1, 128), lambda i, j: (0, 0)),
        ],
        out_specs=pl.BlockSpec((TC_BLOCK_R, TC_BLOCK_C), lambda i, j: (i, j)),
        out_shape=jax.ShapeDtypeStruct((nr - SC_ROWS, nc), jnp.float32),
    )(xt, ctab_tc)

    full = jax.lax.dynamic_update_slice(sc_out, tc_out, (SC_ROWS, 0))
    return full.T
